# baseline (speedup 1.0000x reference)
import functools

import jax
import jax.numpy as jnp
from jax import lax
from jax.experimental import pallas as pl
from jax.experimental.pallas import tpu as pltpu
from jax.experimental.pallas import tpu_sc as plsc

N = 10000
E = 320000
D = 128
DE = 16
DU = 32

NC = 2
NS = 16
NW = NC * NS
CH = 400
SUB = 100
NSUB = CH // SUB
NCHUNK = E // CH
GRP = 5
NGRP = NCHUNK // NW // GRP
N_PAD = 10240
ROWS_PER_TILE = N_PAD // NS


_sc_mesh = plsc.VectorSubcoreMesh(
    core_axis_name="c", subcore_axis_name="s", num_cores=NC, num_subcores=NS
)


@functools.partial(
    pl.kernel,
    out_type=(
        jax.ShapeDtypeStruct((DE, E), jnp.float32),
        jax.ShapeDtypeStruct((NC, N_PAD, DE), jnp.float32),
    ),
    mesh=_sc_mesh,
    scratch_types=[
        pltpu.VMEM((GRP, NSUB, SUB), jnp.int32),
        pltpu.VMEM((GRP, NSUB, SUB), jnp.int32),
        pltpu.VMEM((GRP, DE, CH + 1), jnp.float32),
        pltpu.VMEM((GRP, CH, DE), jnp.float32),
        pltpu.VMEM((GRP, CH, DE), jnp.float32),
        pltpu.VMEM((CH, DE), jnp.float32),
        pltpu.VMEM((DE, CH + 1), jnp.float32),
        pltpu.VMEM((ROWS_PER_TILE // 4, DE), jnp.float32),
        pltpu.VMEM_SHARED((N_PAD, DE), jnp.float32),
        pltpu.SemaphoreType.DMA,
        pltpu.SemaphoreType.DMA,
    ],
    compiler_params=pltpu.CompilerParams(use_tc_tiling_on_sc=False,
                                         needs_layout_passes=False),
)
def _edge_sc(row_hbm, col_hbm, etT_hbm, psrc_hbm, pdst_hbm,
             eoutT_hbm, aggp_hbm,
             idxr_v, idxc_v, etT_v, src_v, dst_v, eo_v, eoT_v, z_v, agg_sh,
             sem1, sem2):
    cid = lax.axis_index("c")
    sid = lax.axis_index("s")
    wid = sid * NC + cid
    lane = lax.iota(jnp.int32, 16)

    QR = ROWS_PER_TILE // 4
    def zero_body(i, _):
        z_v[i] = jnp.zeros((DE,), jnp.float32)
        return 0
    lax.fori_loop(0, QR, zero_body, 0)
    for q in range(4):
        pltpu.sync_copy(z_v, agg_sh.at[pl.ds(sid * ROWS_PER_TILE + q * QR, QR)])
    plsc.subcore_barrier()

    def group_body(g, _):
        chunk0 = wid + g * GRP * NW
        idx_cps = []
        et_cps = []
        for k in range(GRP):
            chunk = chunk0 + k * NW
            idx_cps.append(pltpu.async_copy(
                row_hbm.at[pl.ds(chunk * NSUB, NSUB)], idxr_v.at[k], sem2))
            idx_cps.append(pltpu.async_copy(
                col_hbm.at[pl.ds(chunk * NSUB, NSUB)], idxc_v.at[k], sem2))
            et_cps.append(pltpu.async_copy(
                etT_hbm.at[:, pl.ds(chunk * CH, CH)],
                etT_v.at[k, :, pl.ds(0, CH)], sem2))
        gather_cps = [[] for _ in range(GRP)]
        for k in range(GRP):
            idx_cps[2 * k].wait()
            idx_cps[2 * k + 1].wait()
            for s in range(NSUB):
                gather_cps[k].append(pltpu.async_copy(
                    psrc_hbm.at[idxr_v.at[k, s]],
                    src_v.at[k, pl.ds(s * SUB, SUB)], sem1))
                gather_cps[k].append(pltpu.async_copy(
                    pdst_hbm.at[idxc_v.at[k, s]],
                    dst_v.at[k, pl.ds(s * SUB, SUB)], sem1))
        for k in range(GRP):
            chunk = chunk0 + k * NW
            for cp in gather_cps[k]:
                cp.wait()
            et_cps[k].wait()

            @plsc.parallel_loop(0, CH, 1, unroll=25)
            def _edge(i):
                col = jnp.broadcast_to(i, (16,)).astype(jnp.int32)
                et = plsc.load_gather(etT_v.at[k], [lane, col])
                val = jnp.maximum(src_v[k, i] + dst_v[k, i] + et, 0.0)
                eo_v[i] = val
                plsc.store_scatter(eoT_v, [lane, col], val)

            pltpu.sync_copy(eoT_v.at[:, pl.ds(0, CH)],
                            eoutT_hbm.at[:, pl.ds(chunk * CH, CH)])
            for s in range(NSUB):
                pltpu.sync_copy(
                    eo_v.at[pl.ds(s * SUB, SUB)],
                    agg_sh.at[idxc_v.at[k, s]], add=True)
        return 0

    lax.fori_loop(0, NGRP, group_body, 0)

    plsc.subcore_barrier()
    pltpu.sync_copy(
        agg_sh.at[pl.ds(sid * ROWS_PER_TILE, ROWS_PER_TILE)],
        aggp_hbm.at[cid, pl.ds(sid * ROWS_PER_TILE, ROWS_PER_TILE)],
    )


def _proj_body(node_ref, wsrc_ref, wdst_ref, psrc_ref, pdst_ref):
    x = node_ref[...]
    psrc_ref[...] = jnp.dot(x, wsrc_ref[...], preferred_element_type=jnp.float32)
    pdst_ref[...] = jnp.dot(x, wdst_ref[...], preferred_element_type=jnp.float32)


def _edge_term_body(eat_ref, wt_ref, b_ref, out_ref):
    out_ref[...] = (
        jnp.dot(wt_ref[...], eat_ref[...], preferred_element_type=jnp.float32)
        + b_ref[...]
    )


def _node_body(node_ref, aggp_ref, wv1_ref, wv2_ref, bv_ref,
               vout_ref, vsum_ref, esum_ref):
    i = pl.program_id(0)
    agg = aggp_ref[0] + aggp_ref[1]
    v = jnp.dot(node_ref[...], wv1_ref[...], preferred_element_type=jnp.float32)
    v += jnp.dot(agg, wv2_ref[...], preferred_element_type=jnp.float32)
    v = jnp.maximum(v + bv_ref[...], 0.0)
    vout_ref[...] = v

    @pl.when(i == 0)
    def _():
        vsum_ref[...] = jnp.zeros_like(vsum_ref)
        esum_ref[...] = jnp.zeros_like(esum_ref)
    vsum_ref[...] += jnp.sum(v, axis=0, keepdims=True)
    esum_ref[...] += jnp.sum(agg, axis=0, keepdims=True)


def _global_body(vsum_ref, esum_ref, u_ref, wuv_ref, wue_ref, wuu_ref, bu_ref,
                 out_ref):
    mean_v = vsum_ref[...] / N
    mean_e = esum_ref[...] / E
    acc = jnp.dot(mean_v, wuv_ref[...], preferred_element_type=jnp.float32)
    acc += jnp.dot(mean_e, wue_ref[...], preferred_element_type=jnp.float32)
    acc += jnp.dot(u_ref[...], wuu_ref[...], preferred_element_type=jnp.float32)
    out_ref[...] = jnp.maximum(acc + bu_ref[...], 0.0)


def kernel(node_attr, connectivity, edge_attr, u, W_e, b_e, W_v, b_v, W_u, b_u):
    row = connectivity[0]
    col = connectivity[1]
    row2d = row.reshape(E // SUB, SUB)
    col2d = col.reshape(E // SUB, SUB)
    W_src = W_e[:D]
    W_dst = W_e[D:2 * D]
    W_eg = W_e[2 * D:]
    W_v1 = W_v[:D]
    W_v2 = W_v[D:]
    W_uv = W_u[:D]
    W_ue = W_u[D:D + DE]
    W_uu = W_u[D + DE:]

    eye8 = jnp.eye(8, dtype=jnp.float32)
    psrc_p, pdst_p = pl.pallas_call(
        _proj_body,
        out_shape=(
            jax.ShapeDtypeStruct((N // 8, 128), jnp.float32),
            jax.ShapeDtypeStruct((N // 8, 128), jnp.float32),
        ),
    )(node_attr.reshape(N // 8, 8 * D), jnp.kron(eye8, W_src),
      jnp.kron(eye8, W_dst))
    psrc = psrc_p.reshape(N, DE)
    pdst = pdst_p.reshape(N, DE)

    EB = 16000
    etT = pl.pallas_call(
        _edge_term_body,
        grid=(E // EB,),
        in_specs=[
            pl.BlockSpec((DE, EB), lambda i: (0, i)),
            pl.BlockSpec((DE, DE), lambda i: (0, 0)),
            pl.BlockSpec((DE, 1), lambda i: (0, 0)),
        ],
        out_specs=pl.BlockSpec((DE, EB), lambda i: (0, i)),
        out_shape=jax.ShapeDtypeStruct((DE, E), jnp.float32),
    )(edge_attr.T, W_eg.T, b_e.reshape(DE, 1))

    eoutT, aggp = _edge_sc(row2d, col2d, etT, psrc, pdst)
    e_out = eoutT.T

    NB = 2000
    v_out, vsum, esum = pl.pallas_call(
        _node_body,
        grid=(N // NB,),
        in_specs=[
            pl.BlockSpec((NB, D), lambda i: (i, 0)),
            pl.BlockSpec((NC, NB, DE), lambda i: (0, i, 0)),
            pl.BlockSpec((D, D), lambda i: (0, 0)),
            pl.BlockSpec((DE, D), lambda i: (0, 0)),
            pl.BlockSpec((1, D), lambda i: (0, 0)),
        ],
        out_specs=(
            pl.BlockSpec((NB, D), lambda i: (i, 0)),
            pl.BlockSpec((1, D), lambda i: (0, 0)),
            pl.BlockSpec((1, DE), lambda i: (0, 0)),
        ),
        out_shape=(
            jax.ShapeDtypeStruct((N, D), jnp.float32),
            jax.ShapeDtypeStruct((1, D), jnp.float32),
            jax.ShapeDtypeStruct((1, DE), jnp.float32),
        ),
    )(node_attr, aggp, W_v1, W_v2, b_v.reshape(1, D))

    u_out = pl.pallas_call(
        _global_body,
        out_shape=jax.ShapeDtypeStruct((1, DU), jnp.float32),
    )(vsum, esum, u, W_uv, W_ue, W_uu, b_u.reshape(1, DU))

    return (v_out, e_out, u_out)


# speedup vs baseline: 1.0054x; 1.0054x over previous
import functools

import jax
import jax.numpy as jnp
from jax import lax
from jax.experimental import pallas as pl
from jax.experimental.pallas import tpu as pltpu
from jax.experimental.pallas import tpu_sc as plsc

N = 10000
E = 320000
D = 128
DE = 16
DU = 32

NC = 2
NS = 16
NW = NC * NS
CH = 400
SUB = 100
NSUB = CH // SUB
NCHUNK = E // CH
GRP = 5
NGRP = NCHUNK // NW // GRP
N_PAD = 10240
ROWS_PER_TILE = N_PAD // NS


_sc_mesh = plsc.VectorSubcoreMesh(
    core_axis_name="c", subcore_axis_name="s", num_cores=NC, num_subcores=NS
)


@functools.partial(
    pl.kernel,
    out_type=(
        jax.ShapeDtypeStruct((DE, E), jnp.float32),
        jax.ShapeDtypeStruct((NC, N_PAD, DE), jnp.float32),
    ),
    mesh=_sc_mesh,
    scratch_types=[
        pltpu.VMEM((GRP, NSUB, SUB), jnp.int32),
        pltpu.VMEM((GRP, NSUB, SUB), jnp.int32),
        pltpu.VMEM((GRP, DE, CH + 1), jnp.float32),
        pltpu.VMEM((GRP, CH, DE), jnp.float32),
        pltpu.VMEM((GRP, CH, DE), jnp.float32),
        pltpu.VMEM((CH, DE), jnp.float32),
        pltpu.VMEM((DE, CH + 1), jnp.float32),
        pltpu.VMEM((ROWS_PER_TILE // 4, DE), jnp.float32),
        pltpu.VMEM_SHARED((N_PAD, DE), jnp.float32),
        pltpu.SemaphoreType.DMA,
        pltpu.SemaphoreType.DMA,
    ],
    compiler_params=pltpu.CompilerParams(use_tc_tiling_on_sc=False,
                                         needs_layout_passes=False),
)
def _edge_sc(row_hbm, col_hbm, etT_hbm, psrc_hbm, pdst_hbm,
             eoutT_hbm, aggp_hbm,
             idxr_v, idxc_v, etT_v, src_v, dst_v, eo_v, eoT_v, z_v, agg_sh,
             sem1, sem2):
    cid = lax.axis_index("c")
    sid = lax.axis_index("s")
    wid = sid * NC + cid
    lane = lax.iota(jnp.int32, 16)

    QR = ROWS_PER_TILE // 4
    def zero_body(i, _):
        z_v[i] = jnp.zeros((DE,), jnp.float32)
        return 0
    lax.fori_loop(0, QR, zero_body, 0)
    for q in range(4):
        pltpu.sync_copy(z_v, agg_sh.at[pl.ds(sid * ROWS_PER_TILE + q * QR, QR)])
    plsc.subcore_barrier()

    def group_body(g, _):
        chunk0 = wid + g * GRP * NW
        idx_cps = []
        et_cps = []
        for k in range(GRP):
            chunk = chunk0 + k * NW
            idx_cps.append(pltpu.async_copy(
                row_hbm.at[pl.ds(chunk * NSUB, NSUB)], idxr_v.at[k], sem2))
            idx_cps.append(pltpu.async_copy(
                col_hbm.at[pl.ds(chunk * NSUB, NSUB)], idxc_v.at[k], sem2))
            et_cps.append(pltpu.async_copy(
                etT_hbm.at[:, pl.ds(chunk * CH, CH)],
                etT_v.at[k, :, pl.ds(0, CH)], sem2))
        gather_cps = [[] for _ in range(GRP)]
        for k in range(GRP):
            idx_cps[2 * k].wait()
            idx_cps[2 * k + 1].wait()
            for s in range(NSUB):
                gather_cps[k].append(pltpu.async_copy(
                    psrc_hbm.at[idxr_v.at[k, s]],
                    src_v.at[k, pl.ds(s * SUB, SUB)], sem1))
                gather_cps[k].append(pltpu.async_copy(
                    pdst_hbm.at[idxc_v.at[k, s]],
                    dst_v.at[k, pl.ds(s * SUB, SUB)], sem1))
        for k in range(GRP):
            chunk = chunk0 + k * NW
            for cp in gather_cps[k]:
                cp.wait()
            et_cps[k].wait()

            @plsc.parallel_loop(0, CH, 1, unroll=16)
            def _edge(i):
                col = jnp.broadcast_to(i, (16,)).astype(jnp.int32)
                et = plsc.load_gather(etT_v.at[k], [lane, col])
                val = jnp.maximum(src_v[k, i] + dst_v[k, i] + et, 0.0)
                eo_v[i] = val
                plsc.store_scatter(eoT_v, [lane, col], val)

            pltpu.sync_copy(eoT_v.at[:, pl.ds(0, CH)],
                            eoutT_hbm.at[:, pl.ds(chunk * CH, CH)])
            for s in range(NSUB):
                pltpu.sync_copy(
                    eo_v.at[pl.ds(s * SUB, SUB)],
                    agg_sh.at[idxc_v.at[k, s]], add=True)
        return 0

    lax.fori_loop(0, NGRP, group_body, 0)

    plsc.subcore_barrier()
    pltpu.sync_copy(
        agg_sh.at[pl.ds(sid * ROWS_PER_TILE, ROWS_PER_TILE)],
        aggp_hbm.at[cid, pl.ds(sid * ROWS_PER_TILE, ROWS_PER_TILE)],
    )


def _proj_body(node_ref, wsrc_ref, wdst_ref, psrc_ref, pdst_ref):
    x = node_ref[...]
    psrc_ref[...] = jnp.dot(x, wsrc_ref[...], preferred_element_type=jnp.float32)
    pdst_ref[...] = jnp.dot(x, wdst_ref[...], preferred_element_type=jnp.float32)


def _edge_term_body(eat_ref, wt_ref, b_ref, out_ref):
    out_ref[...] = (
        jnp.dot(wt_ref[...], eat_ref[...], preferred_element_type=jnp.float32)
        + b_ref[...]
    )


def _node_body(node_ref, aggp_ref, wv1_ref, wv2_ref, bv_ref,
               vout_ref, vsum_ref, esum_ref):
    i = pl.program_id(0)
    agg = aggp_ref[0] + aggp_ref[1]
    v = jnp.dot(node_ref[...], wv1_ref[...], preferred_element_type=jnp.float32)
    v += jnp.dot(agg, wv2_ref[...], preferred_element_type=jnp.float32)
    v = jnp.maximum(v + bv_ref[...], 0.0)
    vout_ref[...] = v

    @pl.when(i == 0)
    def _():
        vsum_ref[...] = jnp.zeros_like(vsum_ref)
        esum_ref[...] = jnp.zeros_like(esum_ref)
    vsum_ref[...] += jnp.sum(v, axis=0, keepdims=True)
    esum_ref[...] += jnp.sum(agg, axis=0, keepdims=True)


def _global_body(vsum_ref, esum_ref, u_ref, wuv_ref, wue_ref, wuu_ref, bu_ref,
                 out_ref):
    mean_v = vsum_ref[...] / N
    mean_e = esum_ref[...] / E
    acc = jnp.dot(mean_v, wuv_ref[...], preferred_element_type=jnp.float32)
    acc += jnp.dot(mean_e, wue_ref[...], preferred_element_type=jnp.float32)
    acc += jnp.dot(u_ref[...], wuu_ref[...], preferred_element_type=jnp.float32)
    out_ref[...] = jnp.maximum(acc + bu_ref[...], 0.0)


def kernel(node_attr, connectivity, edge_attr, u, W_e, b_e, W_v, b_v, W_u, b_u):
    row = connectivity[0]
    col = connectivity[1]
    row2d = row.reshape(E // SUB, SUB)
    col2d = col.reshape(E // SUB, SUB)
    W_src = W_e[:D]
    W_dst = W_e[D:2 * D]
    W_eg = W_e[2 * D:]
    W_v1 = W_v[:D]
    W_v2 = W_v[D:]
    W_uv = W_u[:D]
    W_ue = W_u[D:D + DE]
    W_uu = W_u[D + DE:]

    eye8 = jnp.eye(8, dtype=jnp.float32)
    psrc_p, pdst_p = pl.pallas_call(
        _proj_body,
        out_shape=(
            jax.ShapeDtypeStruct((N // 8, 128), jnp.float32),
            jax.ShapeDtypeStruct((N // 8, 128), jnp.float32),
        ),
    )(node_attr.reshape(N // 8, 8 * D), jnp.kron(eye8, W_src),
      jnp.kron(eye8, W_dst))
    psrc = psrc_p.reshape(N, DE)
    pdst = pdst_p.reshape(N, DE)

    EB = 16000
    etT = pl.pallas_call(
        _edge_term_body,
        grid=(E // EB,),
        in_specs=[
            pl.BlockSpec((DE, EB), lambda i: (0, i)),
            pl.BlockSpec((DE, DE), lambda i: (0, 0)),
            pl.BlockSpec((DE, 1), lambda i: (0, 0)),
        ],
        out_specs=pl.BlockSpec((DE, EB), lambda i: (0, i)),
        out_shape=jax.ShapeDtypeStruct((DE, E), jnp.float32),
    )(edge_attr.T, W_eg.T, b_e.reshape(DE, 1))

    eoutT, aggp = _edge_sc(row2d, col2d, etT, psrc, pdst)
    e_out = eoutT.T

    NB = 2000
    v_out, vsum, esum = pl.pallas_call(
        _node_body,
        grid=(N // NB,),
        in_specs=[
            pl.BlockSpec((NB, D), lambda i: (i, 0)),
            pl.BlockSpec((NC, NB, DE), lambda i: (0, i, 0)),
            pl.BlockSpec((D, D), lambda i: (0, 0)),
            pl.BlockSpec((DE, D), lambda i: (0, 0)),
            pl.BlockSpec((1, D), lambda i: (0, 0)),
        ],
        out_specs=(
            pl.BlockSpec((NB, D), lambda i: (i, 0)),
            pl.BlockSpec((1, D), lambda i: (0, 0)),
            pl.BlockSpec((1, DE), lambda i: (0, 0)),
        ),
        out_shape=(
            jax.ShapeDtypeStruct((N, D), jnp.float32),
            jax.ShapeDtypeStruct((1, D), jnp.float32),
            jax.ShapeDtypeStruct((1, DE), jnp.float32),
        ),
    )(node_attr, aggp, W_v1, W_v2, b_v.reshape(1, D))

    u_out = pl.pallas_call(
        _global_body,
        out_shape=jax.ShapeDtypeStruct((1, DU), jnp.float32),
    )(vsum, esum, u, W_uv, W_ue, W_uu, b_u.reshape(1, DU))

    return (v_out, e_out, u_out)
